# baseline (speedup 1.0000x reference)
import jax
import jax.numpy as jnp
from jax import lax
from jax.experimental import pallas as pl
from jax.experimental.pallas import tpu as pltpu, tpu_sc as plsc

N = 100000
F = 128
HIDDEN = 128
VOCAB = 21
OUT = HIDDEN + F - 1
NC = 2
NS = 16
NW = NC * NS
RPW = 3128
CHUNK = 64
NSLOT = 4
_OFFS = list(range(0, RPW - CHUNK, CHUNK)) + [RPW - CHUNK]
_NCH = len(_OFFS)


def _body(x_hbm, emb_hbm, out_hbm, xv, outv, idxi, emb_v, sem_e, sem_x,
          sem_w):
    wid = lax.axis_index("s") * NC + lax.axis_index("c")
    base = pl.multiple_of(jnp.minimum(wid * RPW, N - RPW), 8)
    lanes = lax.iota(jnp.int32, 16)

    pltpu.sync_copy(emb_hbm, emb_v)

    def xcp(i):
        r0 = pl.multiple_of((base + _OFFS[i]) * F, 8)
        return pltpu.make_async_copy(
            x_hbm.at[pl.ds(r0, CHUNK * F)], xv[i % NSLOT].at[pl.ds(0, CHUNK * F)],
            sem_x)

    def wcp(i):
        r0 = pl.multiple_of((base + _OFFS[i]) * OUT, 8)
        return pltpu.make_async_copy(
            outv[i % NSLOT].at[pl.ds(0, CHUNK * OUT)],
            out_hbm.at[pl.ds(r0, CHUNK * OUT)], sem_w)

    for j in range(min(NSLOT, _NCH)):
        xcp(j).start()
    for i in range(_NCH):
        if i >= NSLOT:
            wcp(i - NSLOT).wait()
        xcp(i).wait()
        xvb = xv[i % NSLOT]
        ovb = outv[i % NSLOT]
        for g in range(0):
            v = plsc.load_gather(xvb, [lanes * F + g * 16 * F])
            idxi[pl.ds(g * 16, 16)] = v.astype(jnp.int32)

        tail_mask = lanes < 15

        @plsc.parallel_loop(0, 0, step=1, unroll=4)
        def row(r):
            idxv = plsc.load_gather(idxi, [jnp.full((16,), 0, jnp.int32) + r])
            ebase = idxv * HIDDEN + lanes
            obase = r * OUT + lanes
            for k in range(HIDDEN // 16):
                vals = plsc.load_gather(emb_v, [ebase + k * 16])
                plsc.store_scatter(ovb, [obase + k * 16], vals)
            pbase = r * F + 1 + lanes
            for k in range(F // 16):
                vals = plsc.load_gather(xvb, [pbase + k * 16])
                if k == F // 16 - 1:
                    plsc.store_scatter(ovb, [obase + HIDDEN + k * 16], vals,
                                       mask=tail_mask)
                else:
                    plsc.store_scatter(ovb, [obase + HIDDEN + k * 16], vals)
        wcp(i).start()
        if i + NSLOT < _NCH:
            xcp(i + NSLOT).start()
    for j in range(max(0, _NCH - NSLOT), _NCH):
        wcp(j).wait()


def kernel(x, emb):
    mesh = plsc.VectorSubcoreMesh(core_axis_name="c", subcore_axis_name="s")
    run = pl.kernel(
        _body,
        out_type=jax.ShapeDtypeStruct((N * OUT,), jnp.float32),
        mesh=mesh,
        compiler_params=pltpu.CompilerParams(
            use_tc_tiling_on_sc=False, needs_layout_passes=False),
        scratch_types=[
            [pltpu.VMEM((CHUNK * F + 16,), jnp.float32) for _ in range(NSLOT)],
            [pltpu.VMEM((CHUNK * OUT + 16,), jnp.float32)
             for _ in range(NSLOT)],
            pltpu.VMEM((CHUNK,), jnp.int32),
            pltpu.VMEM((VOCAB * HIDDEN,), jnp.float32),
            pltpu.SemaphoreType.DMA,
            pltpu.SemaphoreType.DMA,
            pltpu.SemaphoreType.DMA,
        ],
    )
    out_flat = run(x.reshape(-1), emb.reshape(-1))
    return out_flat.reshape(N, OUT)


# speedup vs baseline: 3.3302x; 3.3302x over previous
import jax
import jax.numpy as jnp
from jax.experimental import pallas as pl


BLOCK_N = 2000
VOCAB = 21
VOCAB_PAD = 32


def _body(x_ref, emb_ref, out_ref):
    xb = x_ref[...]
    idx = xb[:, 0].astype(jnp.int32)
    classes = jax.lax.broadcasted_iota(jnp.int32, (xb.shape[0], VOCAB_PAD), 1)
    onehot = (idx[:, None] == classes).astype(jnp.float32)
    enc = jnp.dot(onehot, emb_ref[...], preferred_element_type=jnp.float32)
    out_ref[:, :128] = enc
    out_ref[:, 128:] = xb[:, 1:]


def kernel(x, emb):
    n, f = x.shape
    hidden = emb.shape[1]
    emb_p = jnp.pad(emb, ((0, VOCAB_PAD - emb.shape[0]), (0, 0)))
    grid = (n // BLOCK_N,)
    return pl.pallas_call(
        _body,
        grid=grid,
        in_specs=[
            pl.BlockSpec((BLOCK_N, f), lambda i: (i, 0)),
            pl.BlockSpec((VOCAB_PAD, hidden), lambda i: (0, 0)),
        ],
        out_specs=pl.BlockSpec((BLOCK_N, hidden + f - 1), lambda i: (i, 0)),
        out_shape=jax.ShapeDtypeStruct((n, hidden + f - 1), jnp.float32),
    )(x, emb_p)


# speedup vs baseline: 4.1754x; 1.2538x over previous
import jax
import jax.numpy as jnp
from jax.experimental import pallas as pl


BLOCK_N = 4000
VOCAB = 21
VOCAB_PAD = 32


def _body(x_ref, emb_ref, out_ref):
    xb = x_ref[...]
    idx = xb[:, 0].astype(jnp.int32)
    classes = jax.lax.broadcasted_iota(jnp.int32, (xb.shape[0], VOCAB_PAD), 1)
    onehot = (idx[:, None] == classes).astype(jnp.float32)
    enc = jnp.dot(onehot, emb_ref[...], preferred_element_type=jnp.float32)
    out_ref[:, :128] = enc
    out_ref[:, 128:] = xb[:, 1:]


def kernel(x, emb):
    n, f = x.shape
    hidden = emb.shape[1]
    emb_p = jnp.pad(emb, ((0, VOCAB_PAD - emb.shape[0]), (0, 0)))
    grid = (n // BLOCK_N,)
    return pl.pallas_call(
        _body,
        grid=grid,
        in_specs=[
            pl.BlockSpec((BLOCK_N, f), lambda i: (i, 0)),
            pl.BlockSpec((VOCAB_PAD, hidden), lambda i: (0, 0)),
        ],
        out_specs=pl.BlockSpec((BLOCK_N, hidden + f - 1), lambda i: (i, 0)),
        out_shape=jax.ShapeDtypeStruct((n, hidden + f - 1), jnp.float32),
    )(x, emb_p)


# speedup vs baseline: 4.5797x; 1.0968x over previous
import jax
import jax.numpy as jnp
from jax.experimental import pallas as pl


BLOCK_N = 10000
VOCAB = 21
VOCAB_PAD = 32


def _body(x_ref, emb_ref, out_ref):
    xb = x_ref[...]
    idx = xb[:, 0].astype(jnp.int32)
    classes = jax.lax.broadcasted_iota(jnp.int32, (xb.shape[0], VOCAB_PAD), 1)
    onehot = (idx[:, None] == classes).astype(jnp.float32)
    enc = jnp.dot(onehot, emb_ref[...], preferred_element_type=jnp.float32)
    out_ref[:, :128] = enc
    out_ref[:, 128:] = xb[:, 1:]


def kernel(x, emb):
    n, f = x.shape
    hidden = emb.shape[1]
    emb_p = jnp.pad(emb, ((0, VOCAB_PAD - emb.shape[0]), (0, 0)))
    grid = (n // BLOCK_N,)
    return pl.pallas_call(
        _body,
        grid=grid,
        in_specs=[
            pl.BlockSpec((BLOCK_N, f), lambda i: (i, 0)),
            pl.BlockSpec((VOCAB_PAD, hidden), lambda i: (0, 0)),
        ],
        out_specs=pl.BlockSpec((BLOCK_N, hidden + f - 1), lambda i: (i, 0)),
        out_shape=jax.ShapeDtypeStruct((n, hidden + f - 1), jnp.float32),
    )(x, emb_p)


# speedup vs baseline: 4.6791x; 1.0217x over previous
import jax
import jax.numpy as jnp
from jax.experimental import pallas as pl


BLOCK_N = 16000
VOCAB = 21
VOCAB_PAD = 32


def _body(x_ref, emb_ref, out_ref):
    xb = x_ref[...]
    idx = xb[:, 0].astype(jnp.int32)
    classes = jax.lax.broadcasted_iota(jnp.int32, (xb.shape[0], VOCAB_PAD), 1)
    onehot = (idx[:, None] == classes).astype(jnp.float32)
    enc = jnp.dot(onehot, emb_ref[...], preferred_element_type=jnp.float32)
    out_ref[:, :128] = enc
    out_ref[:, 128:] = xb[:, 1:]


def kernel(x, emb):
    n, f = x.shape
    hidden = emb.shape[1]
    emb_p = jnp.pad(emb, ((0, VOCAB_PAD - emb.shape[0]), (0, 0)))
    grid = (pl.cdiv(n, BLOCK_N),)
    return pl.pallas_call(
        _body,
        grid=grid,
        in_specs=[
            pl.BlockSpec((BLOCK_N, f), lambda i: (i, 0)),
            pl.BlockSpec((VOCAB_PAD, hidden), lambda i: (0, 0)),
        ],
        out_specs=pl.BlockSpec((BLOCK_N, hidden + f - 1), lambda i: (i, 0)),
        out_shape=jax.ShapeDtypeStruct((n, hidden + f - 1), jnp.float32),
    )(x, emb_p)
